# initial kernel scaffold (unmeasured)
import jax
import jax.numpy as jnp
from jax import lax
from jax.experimental import pallas as pl
from jax.experimental.pallas import tpu as pltpu

M_HALF = 2048
BM, BN, BK = 1024, 2048, 1024


def _gemm_body(dy_ref, w_ref, out_ref):
    @pl.when(pl.program_id(2) == 0)
    def _():
        out_ref[...] = jnp.zeros_like(out_ref)

    out_ref[...] += lax.dot_general(
        dy_ref[...],
        w_ref[...],
        (((1,), (1,)), ((), ())),
        preferred_element_type=jnp.float32,
    )


def _local_quarter_gemm(dy_half, w):
    m, k = dy_half.shape
    n = w.shape[0]
    return pl.pallas_call(
        _gemm_body,
        grid=(m // BM, n // BN, k // BK),
        in_specs=[
            pl.BlockSpec((BM, BK), lambda i, j, kk: (i, kk)),
            pl.BlockSpec((BN, BK), lambda i, j, kk: (j, kk)),
        ],
        out_specs=pl.BlockSpec((BM, BN), lambda i, j, kk: (i, j)),
        out_shape=jax.ShapeDtypeStruct((m, n), jnp.float32),
    )(dy_half, w)


def _exchange_y_body(q_ref, r_ref, recv_ref, send_sem, recv_sem):
    my_x = lax.axis_index("x")
    my_y = lax.axis_index("y")
    nbr = (my_x, 1 - my_y)

    barrier = pltpu.get_barrier_semaphore()
    pl.semaphore_signal(
        barrier, inc=1, device_id=nbr, device_id_type=pl.DeviceIdType.MESH
    )
    pl.semaphore_wait(barrier, 1)

    rdma = pltpu.make_async_remote_copy(
        src_ref=q_ref,
        dst_ref=recv_ref,
        send_sem=send_sem,
        recv_sem=recv_sem,
        device_id=nbr,
        device_id_type=pl.DeviceIdType.MESH,
    )
    rdma.start()
    rdma.wait()
    r_ref[...] = q_ref[...] + recv_ref[...]


def _exchange_y(q):
    return pl.pallas_call(
        _exchange_y_body,
        out_shape=jax.ShapeDtypeStruct(q.shape, q.dtype),
        in_specs=[pl.BlockSpec(memory_space=pltpu.VMEM)],
        out_specs=pl.BlockSpec(memory_space=pltpu.VMEM),
        scratch_shapes=[
            pltpu.VMEM(q.shape, q.dtype),
            pltpu.SemaphoreType.DMA,
            pltpu.SemaphoreType.DMA,
        ],
        compiler_params=pltpu.CompilerParams(collective_id=0),
    )(q)


def _concat_x_body(r_ref, out_ref, send_sem, recv_sem):
    my_x = lax.axis_index("x")
    my_y = lax.axis_index("y")
    nbr = (1 - my_x, my_y)

    barrier = pltpu.get_barrier_semaphore()
    pl.semaphore_signal(
        barrier, inc=1, device_id=nbr, device_id_type=pl.DeviceIdType.MESH
    )
    pl.semaphore_wait(barrier, 1)

    rdma = pltpu.make_async_remote_copy(
        src_ref=r_ref,
        dst_ref=out_ref.at[pl.ds(my_x * M_HALF, M_HALF)],
        send_sem=send_sem,
        recv_sem=recv_sem,
        device_id=nbr,
        device_id_type=pl.DeviceIdType.MESH,
    )
    rdma.start()
    out_ref[pl.ds(my_x * M_HALF, M_HALF), :] = r_ref[...]
    rdma.wait()


def _concat_x(r):
    m_half, n = r.shape
    return pl.pallas_call(
        _concat_x_body,
        out_shape=jax.ShapeDtypeStruct((2 * m_half, n), r.dtype),
        in_specs=[pl.BlockSpec(memory_space=pltpu.VMEM)],
        out_specs=pl.BlockSpec(memory_space=pltpu.VMEM),
        scratch_shapes=[
            pltpu.SemaphoreType.DMA,
            pltpu.SemaphoreType.DMA,
        ],
        compiler_params=pltpu.CompilerParams(collective_id=1),
    )(r)


def kernel(dy, W):
    my_x = lax.axis_index("x")
    dy_half = lax.dynamic_slice_in_dim(dy, my_x * M_HALF, M_HALF, axis=0)
    q = _local_quarter_gemm(dy_half, W)
    r = _exchange_y(q)
    return _concat_x(r)


# baseline (device time: 1018371 ns/iter reference)
import jax
import jax.numpy as jnp
from jax import lax
from jax.experimental import pallas as pl
from jax.experimental.pallas import tpu as pltpu

M_HALF = 2048
N_OUT = 4096
CH = 256
NC = M_HALF // CH
BM, BN, BK = 1024, 2048, 1024


def _gemm_body(dy_ref, w_ref, out_ref):
    @pl.when(pl.program_id(2) == 0)
    def _():
        out_ref[...] = jnp.zeros_like(out_ref)

    out_ref[...] += lax.dot_general(
        dy_ref[...],
        w_ref[...],
        (((1,), (1,)), ((), ())),
        preferred_element_type=jnp.float32,
    )


def _local_quarter_gemm(dy_half, w):
    m, k = dy_half.shape
    n = w.shape[0]
    return pl.pallas_call(
        _gemm_body,
        grid=(m // BM, n // BN, k // BK),
        in_specs=[
            pl.BlockSpec((BM, BK), lambda i, j, kk: (i, kk)),
            pl.BlockSpec((BN, BK), lambda i, j, kk: (j, kk)),
        ],
        out_specs=pl.BlockSpec((BM, BN), lambda i, j, kk: (i, j)),
        out_shape=jax.ShapeDtypeStruct((m, n), jnp.float32),
        compiler_params=pltpu.CompilerParams(vmem_limit_bytes=60 * 1024 * 1024),
    )(dy_half, w)


def _comm_body(
    q_ref,
    out_ref,
    q_vm,
    recv_vm,
    r_vm,
    sem_q,
    send_y,
    recv_y,
    send_x,
    recv_x,
    sem_store,
    credit_y,
    credit_x,
):
    my_x = lax.axis_index("x")
    my_y = lax.axis_index("y")
    ynbr = (my_x, 1 - my_y)
    xnbr = (1 - my_x, my_y)

    barrier = pltpu.get_barrier_semaphore()
    for nbr in (ynbr, xnbr):
        pl.semaphore_signal(
            barrier, inc=1, device_id=nbr, device_id_type=pl.DeviceIdType.MESH
        )
    pl.semaphore_wait(barrier, 2)

    my_base = my_x * M_HALF
    other_base = (1 - my_x) * M_HALF

    rdmas_y = []
    rdmas_x = []
    stores = []
    for c in range(NC):
        slot = c % 2
        if c >= 2:
            rdmas_y[c - 2].wait_send()
            pl.semaphore_wait(credit_y.at[slot], 1)
        rdma_y = pltpu.make_async_remote_copy(
            src_ref=q_ref.at[pl.ds(c * CH, CH)],
            dst_ref=recv_vm.at[slot],
            send_sem=send_y.at[slot],
            recv_sem=recv_y.at[slot],
            device_id=ynbr,
            device_id_type=pl.DeviceIdType.MESH,
        )
        rdma_y.start()
        rdmas_y.append(rdma_y)
        cp_q = pltpu.make_async_copy(
            q_ref.at[pl.ds(c * CH, CH)], q_vm.at[slot], sem_q.at[slot]
        )
        cp_q.start()

        if c >= 2:
            rdmas_x[c - 2].wait_send()
            stores[c - 2].wait()
            pl.semaphore_wait(credit_x.at[slot], 1)

        rdma_y.wait_recv()
        cp_q.wait()
        r_vm[slot] = q_vm[slot] + recv_vm[slot]
        if c <= NC - 3:
            pl.semaphore_signal(
                credit_y.at[slot],
                inc=1,
                device_id=ynbr,
                device_id_type=pl.DeviceIdType.MESH,
            )

        rdma_x = pltpu.make_async_remote_copy(
            src_ref=r_vm.at[slot],
            dst_ref=out_ref.at[pl.ds(my_base + c * CH, CH)],
            send_sem=send_x.at[slot],
            recv_sem=recv_x.at[slot],
            device_id=xnbr,
            device_id_type=pl.DeviceIdType.MESH,
        )
        rdma_x.start()
        rdmas_x.append(rdma_x)
        cp_out = pltpu.make_async_copy(
            r_vm.at[slot],
            out_ref.at[pl.ds(my_base + c * CH, CH)],
            sem_store.at[slot],
        )
        cp_out.start()
        stores.append(cp_out)

        rdma_in = pltpu.make_async_remote_copy(
            src_ref=r_vm.at[slot],
            dst_ref=out_ref.at[pl.ds(other_base + c * CH, CH)],
            send_sem=send_x.at[slot],
            recv_sem=recv_x.at[slot],
            device_id=xnbr,
            device_id_type=pl.DeviceIdType.MESH,
        )
        rdma_in.wait_recv()
        if c <= NC - 3:
            pl.semaphore_signal(
                credit_x.at[slot],
                inc=1,
                device_id=xnbr,
                device_id_type=pl.DeviceIdType.MESH,
            )

    for c in (NC - 2, NC - 1):
        rdmas_y[c].wait_send()
        rdmas_x[c].wait_send()
        stores[c].wait()


def _comm(q):
    return pl.pallas_call(
        _comm_body,
        out_shape=jax.ShapeDtypeStruct((2 * M_HALF, N_OUT), q.dtype),
        in_specs=[pl.BlockSpec(memory_space=pltpu.MemorySpace.HBM)],
        out_specs=pl.BlockSpec(memory_space=pltpu.MemorySpace.HBM),
        scratch_shapes=[
            pltpu.VMEM((2, CH, N_OUT), jnp.float32),
            pltpu.VMEM((2, CH, N_OUT), jnp.float32),
            pltpu.VMEM((2, CH, N_OUT), jnp.float32),
            pltpu.SemaphoreType.DMA((2,)),
            pltpu.SemaphoreType.DMA((2,)),
            pltpu.SemaphoreType.DMA((2,)),
            pltpu.SemaphoreType.DMA((2,)),
            pltpu.SemaphoreType.DMA((2,)),
            pltpu.SemaphoreType.DMA((2,)),
            pltpu.SemaphoreType.REGULAR((2,)),
            pltpu.SemaphoreType.REGULAR((2,)),
        ],
        compiler_params=pltpu.CompilerParams(
            collective_id=0, vmem_limit_bytes=60 * 1024 * 1024
        ),
    )(q)


def kernel(dy, W):
    my_x = lax.axis_index("x")
    dy_half = lax.dynamic_slice_in_dim(dy, my_x * M_HALF, M_HALF, axis=0)
    q = _local_quarter_gemm(dy_half, W)
    return _comm(q)
